# baseline (speedup 1.0000x reference)
import functools

import jax
import jax.numpy as jnp
from jax import lax
from jax.experimental import pallas as pl
from jax.experimental.pallas import tpu as pltpu
from jax.experimental.pallas import tpu_sc as plsc

N = 10000
E = 320000
D = 128
ED = 16
L = 3
G = 64

NT = 32
EPT = E // NT
CK = 125
CH = EPT // CK
NOCT = CH // 8
NPAD = 10016
RPS = NPAD // 16

PERM = [32 * g + 2 * i + h for g in range(4) for h in range(2)
        for i in range(16)]

NB = 2000
NGRID = N // NB
BE = 2000


def _edge_emb_body(ea_ref, w_ref, b_ref, o_ref):
    o_ref[...] = (
        jnp.dot(ea_ref[...], w_ref[0], preferred_element_type=jnp.float32)
        + b_ref[...]
    )


def _edge_emb(edge_attr, w, b):
    return pl.pallas_call(
        _edge_emb_body,
        grid=(E // BE,),
        in_specs=[
            pl.BlockSpec((BE, ED), lambda e: (e, 0)),
            pl.BlockSpec((1, ED, D), lambda e: (0, 0, 0)),
            pl.BlockSpec((1, D), lambda e: (0, 0)),
        ],
        out_specs=pl.BlockSpec((BE, D), lambda e: (e, 0)),
        out_shape=jax.ShapeDtypeStruct((E, D), jnp.float32),
    )(edge_attr, w, b)


def _prep_body(x_ref, vn_ref, o_ref, o16_ref):
    h = x_ref[...] + vn_ref[0][None, :]
    o_ref[...] = h
    o16_ref[...] = h.astype(jnp.bfloat16)


def _prep(x, vn_emb):
    return pl.pallas_call(
        _prep_body,
        grid=(NGRID,),
        in_specs=[
            pl.BlockSpec((NB, D), lambda n: (n, 0)),
            pl.BlockSpec((L, D), lambda n: (0, 0)),
        ],
        out_specs=[pl.BlockSpec((NB, D), lambda n: (n, 0))] * 2,
        out_shape=[
            jax.ShapeDtypeStruct((N, D), jnp.float32),
            jax.ShapeDtypeStruct((N, D), jnp.bfloat16),
        ],
    )(x, vn_emb)


_SC_MESH = plsc.VectorSubcoreMesh(core_axis_name="c", subcore_axis_name="s")


@functools.partial(
    pl.kernel,
    out_type=jax.ShapeDtypeStruct((2, NPAD, D), jnp.float32),
    mesh=_SC_MESH,
    compiler_params=pltpu.CompilerParams(use_tc_tiling_on_sc=False,
                                         needs_layout_passes=False),
    scratch_types=[
        pltpu.VMEM((4, 2, 2, CK), jnp.int32),
        pltpu.VMEM((2, CK, D // 2), jnp.int32),
        pltpu.VMEM((2, CK, D), jnp.float32),
        pltpu.VMEM_SHARED((NPAD, D), jnp.float32),
        pltpu.SemaphoreType.DMA,
        pltpu.SemaphoreType.DMA,
        pltpu.SemaphoreType.DMA,
        pltpu.SemaphoreType.DMA,
        pltpu.SemaphoreType.DMA,
        pltpu.SemaphoreType.DMA,
    ],
)
def _sc_aggregate(h16_hbm, eemb_hbm, rec_hbm, zero_hbm, out_hbm,
                  rec_v, rows_v, emb_v, acc,
                  gs0, gs1, ss0, ss1, es0, es1):
    c = lax.axis_index("c")
    s = lax.axis_index("s")
    tile = c * 16 + s
    gsem = (gs0, gs1)
    ssem = (ss0, ss1)
    esem = (es0, es1)
    ebase = tile * EPT

    def rec_load(p, pb):
        pltpu.sync_copy(rec_hbm.at[tile, p], rec_v.at[pb])

    def gather_start(pb, w, rb):
        pltpu.async_copy(h16_hbm.at[rec_v.at[pb, w, 0]], rows_v.at[rb],
                         gsem[rb])

    def gather_wait(pb, w, rb):
        pltpu.make_async_copy(h16_hbm.at[rec_v.at[pb, w, 0]], rows_v.at[rb],
                              gsem[rb]).wait()

    def emb_start(j, eb):
        pltpu.async_copy(eemb_hbm.at[pl.ds(ebase + j * CK, CK)],
                         emb_v.at[eb], esem[eb])

    def emb_wait(j, eb):
        pltpu.make_async_copy(eemb_hbm.at[pl.ds(ebase + j * CK, CK)],
                              emb_v.at[eb], esem[eb]).wait()

    def scatter_start(pb, w, eb):
        pltpu.async_copy(emb_v.at[eb], acc.at[rec_v.at[pb, w, 1]],
                         ssem[eb], add=True)

    def scatter_drain(pb, w, eb):
        pltpu.make_async_copy(emb_v.at[eb], acc.at[rec_v.at[pb, w, 1]],
                              ssem[eb]).wait()

    rec_load(0, 0)
    rec_load(1, 1)
    gather_start(0, 0, 0)
    gather_start(0, 1, 1)
    emb_start(0, 0)
    pltpu.sync_copy(zero_hbm.at[pl.ds(s * RPS, RPS)],
                    acc.at[pl.ds(s * RPS, RPS)])
    plsc.subcore_barrier()

    def oct_body(jj, carry):
        for q in range(8):
            j = jj * 8 + q
            rb = q % 2
            nb = (q + 1) % 2
            pb_j = (q // 2) % 4
            w_j = q % 2
            if q % 2 == 0:
                pb_p = ((q + 6) // 2) % 4
                w_p = 1
            else:
                pb_p = ((q - 1) // 2) % 4
                w_p = 0
            if q > 0:
                scatter_drain(pb_p, w_p, nb)
            else:
                @pl.when(jj > 0)
                def _():
                    scatter_drain(pb_p, w_p, nb)
            if q == 7:
                @pl.when(jj < NOCT - 1)
                def _():
                    emb_start(j + 1, nb)
            else:
                emb_start(j + 1, nb)
            gather_wait(pb_j, w_j, rb)
            emb_wait(j, rb)

            def row_body(r, carry2):
                for g in range(D // 32):
                    hw = rows_v[rb, r, pl.ds(g * 16, 16)]
                    av = plsc.bitcast(hw << 16, jnp.float32)
                    bv = plsc.bitcast(hw & jnp.int32(-65536), jnp.float32)
                    sa = pl.ds(g * 32, 16)
                    sb = pl.ds(g * 32 + 16, 16)
                    emb_v[rb, r, sa] = jnp.maximum(av + emb_v[rb, r, sa], 0.0)
                    emb_v[rb, r, sb] = jnp.maximum(bv + emb_v[rb, r, sb], 0.0)
                return carry2

            lax.fori_loop(0, CK, row_body, 0, unroll=5)
            if q >= 6:
                @pl.when(jj < NOCT - 1)
                def _():
                    gather_start(((q + 2) // 2) % 4, w_j, rb)
            else:
                gather_start(((q + 2) // 2) % 4, w_j, rb)
            scatter_start(pb_j, w_j, rb)
            if q % 2 == 1:
                if q >= 5:
                    @pl.when(jj < NOCT - 1)
                    def _():
                        rec_load(4 * jj + (q + 3) // 2, ((q + 3) // 2) % 4)
                else:
                    rec_load(4 * jj + (q + 3) // 2, ((q + 3) // 2) % 4)
        return carry

    lax.fori_loop(0, NOCT, oct_body, 0, unroll=False)
    scatter_drain(3, 1, 1)
    plsc.subcore_barrier()
    pltpu.sync_copy(acc.at[pl.ds(s * RPS, RPS)],
                    out_hbm.at[c, pl.ds(s * RPS, RPS)])


def _layer_body(h_ref, agg_ref, w1_ref, w1p_ref, b1_ref, w2_ref, b2_ref,
                vn_ref, bt_ref, hout_ref, h16_ref, g_ref):
    aggp = agg_ref[0] + agg_ref[1]
    z = jnp.maximum(
        jnp.dot(h_ref[...], w1_ref[...], preferred_element_type=jnp.float32)
        + jnp.dot(aggp, w1p_ref[...], preferred_element_type=jnp.float32)
        + b1_ref[...], 0.0)
    h_new = (jnp.dot(z, w2_ref[...], preferred_element_type=jnp.float32)
             + b2_ref[...])
    h_cur = h_new + vn_ref[...]
    hout_ref[...] = h_cur
    h16_ref[...] = h_cur.astype(jnp.bfloat16)
    onehot = (bt_ref[0, 0, :][None, :]
              == lax.broadcasted_iota(jnp.int32, (G, NB), 0)
              ).astype(jnp.float32)

    @pl.when(pl.program_id(0) == 0)
    def _():
        g_ref[...] = jnp.zeros_like(g_ref)

    g_ref[...] += jnp.dot(onehot, h_new, preferred_element_type=jnp.float32)


def _layer(h_cur, agg2, w1, w1p, b1, w2, b2, vn_next, batch2):
    return pl.pallas_call(
        _layer_body,
        grid=(NGRID,),
        in_specs=[
            pl.BlockSpec((NB, D), lambda n: (n, 0)),
            pl.BlockSpec((2, NB, D), lambda n: (0, n, 0)),

            pl.BlockSpec((D, D), lambda n: (0, 0)),
            pl.BlockSpec((D, D), lambda n: (0, 0)),
            pl.BlockSpec((1, D), lambda n: (0, 0)),
            pl.BlockSpec((D, D), lambda n: (0, 0)),
            pl.BlockSpec((1, D), lambda n: (0, 0)),
            pl.BlockSpec((1, D), lambda n: (0, 0)),
            pl.BlockSpec((1, 1, NB), lambda n: (n, 0, 0)),
        ],
        out_specs=[
            pl.BlockSpec((NB, D), lambda n: (n, 0)),
            pl.BlockSpec((NB, D), lambda n: (n, 0)),
            pl.BlockSpec((G, D), lambda n: (0, 0)),
        ],
        out_shape=[
            jax.ShapeDtypeStruct((N, D), jnp.float32),
            jax.ShapeDtypeStruct((N, D), jnp.bfloat16),
            jax.ShapeDtypeStruct((G, D), jnp.float32),
        ],
    )(h_cur, agg2, w1, w1p, b1, w2, b2, vn_next, batch2)


def _layer_norm(h, g, b):
    mu = jnp.mean(h, axis=-1, keepdims=True)
    var = jnp.mean((h - mu) ** 2, axis=-1, keepdims=True)
    return (h - mu) * lax.rsqrt(var + 1e-5) * g + b


def _final_body(g0_ref, g1_ref, g2_ref, w0_ref, b0_ref, ln_g0_ref, ln_b0_ref,
                w1_ref, b1_ref, ln_g1_ref, ln_b1_ref, we_ref, be_ref,
                wo_ref, bo_ref, out_ref):
    q = (jnp.dot(g0_ref[...], w0_ref[pl.ds(0, D), :],
                 preferred_element_type=jnp.float32)
         + jnp.dot(g1_ref[...], w0_ref[pl.ds(D, D), :],
                   preferred_element_type=jnp.float32)
         + jnp.dot(g2_ref[...], w0_ref[pl.ds(2 * D, D), :],
                   preferred_element_type=jnp.float32)
         + b0_ref[...])
    q = jnp.maximum(_layer_norm(q, ln_g0_ref[...], ln_b0_ref[...]), 0.0)
    q = jnp.dot(q, w1_ref[...], preferred_element_type=jnp.float32) + b1_ref[...]
    q = jnp.maximum(_layer_norm(q, ln_g1_ref[...], ln_b1_ref[...]), 0.0)
    emb = jnp.dot(q, we_ref[...], preferred_element_type=jnp.float32) + be_ref[...]
    out_ref[...] = (jnp.dot(emb, wo_ref[...], preferred_element_type=jnp.float32)
                    + bo_ref[...])


def _final(g0, g1, g2, w0, b0, ln_g0, ln_b0, w1, b1, ln_g1, ln_b1,
           we, be, wo, bo):
    return pl.pallas_call(
        _final_body,
        out_shape=jax.ShapeDtypeStruct((G, 1), jnp.float32),
    )(g0, g1, g2, w0, b0, ln_g0, ln_b0, w1, b1, ln_g1, ln_b1, we, be, wo, bo)


def kernel(x, edge_index, edge_attr, batch, vn_emb, vn_W1, vn_b1, vn_g, vn_bt,
           vn_W2, vn_b2, agg_eW, agg_eb, agg_W1, agg_b1, agg_W2, agg_b2,
           lin_W0, lin_b0, lin_g0, lin_bt0, lin_W1, lin_b1, lin_g1, lin_bt1,
           emb_W, emb_b, out_W, out_b):
    src_r = edge_index[0].reshape(NT, CH // 2, 2, 1, CK).astype(jnp.int32)
    dst_r = edge_index[1].reshape(NT, CH // 2, 2, 1, CK).astype(jnp.int32)
    rec = jnp.concatenate([src_r, dst_r], axis=3)
    batch2 = batch.reshape(NGRID, 1, NB).astype(jnp.int32)
    zero_nd = jnp.zeros((NPAD, D), jnp.float32)
    perm = jnp.array(PERM, jnp.int32)

    def pack_rows(h16):
        return lax.bitcast_convert_type(h16.reshape(N, D // 2, 2), jnp.int32)

    eW_p = agg_eW[:, :, perm]
    eb_p = agg_eb[:, perm]
    eemb_i = _edge_emb(edge_attr, eW_p[0:1], eb_p[0:1])
    h_cur, h16 = _prep(x, vn_emb)

    gs = []
    for i in range(L):
        agg2 = _sc_aggregate(pack_rows(h16), eemb_i, rec, zero_nd)
        if i + 1 < L:
            eemb_i = _edge_emb(edge_attr, eW_p[i + 1:i + 2],
                               eb_p[i + 1:i + 2])
        if i + 1 < L:
            vn_next = vn_emb[i + 1].reshape(1, D)
        else:
            vn_next = jnp.zeros((1, D), jnp.float32)
        h_cur, h16, g_i = _layer(h_cur, agg2, agg_W1[i], agg_W1[i][perm, :],
                                 agg_b1[i].reshape(1, D), agg_W2[i],
                                 agg_b2[i].reshape(1, D), vn_next, batch2)
        gs.append(g_i)

    return _final(gs[0], gs[1], gs[2], lin_W0, lin_b0.reshape(1, D),
                  lin_g0.reshape(1, D), lin_bt0.reshape(1, D), lin_W1,
                  lin_b1.reshape(1, D), lin_g1.reshape(1, D),
                  lin_bt1.reshape(1, D), emb_W, emb_b.reshape(1, D),
                  out_W, out_b.reshape(1, 1))


# speedup vs baseline: 1.0127x; 1.0127x over previous
import functools

import jax
import jax.numpy as jnp
from jax import lax
from jax.experimental import pallas as pl
from jax.experimental.pallas import tpu as pltpu
from jax.experimental.pallas import tpu_sc as plsc

N = 10000
E = 320000
D = 128
ED = 16
L = 3
G = 64

NT = 32
EPT = E // NT
CK = 125
CH = EPT // CK
NOCT = CH // 8
NPAD = 10016
RPS = NPAD // 16

PERM = [32 * g + 2 * i + h for g in range(4) for h in range(2)
        for i in range(16)]

NB = 2000
NGRID = N // NB
BE = 2000


def _edge_emb_body(ea_ref, w_ref, b_ref, o_ref):
    o_ref[...] = (
        jnp.dot(ea_ref[...], w_ref[0], preferred_element_type=jnp.float32)
        + b_ref[...]
    )


def _edge_emb(edge_attr, w, b):
    return pl.pallas_call(
        _edge_emb_body,
        grid=(E // BE,),
        in_specs=[
            pl.BlockSpec((BE, ED), lambda e: (e, 0)),
            pl.BlockSpec((1, ED, D), lambda e: (0, 0, 0)),
            pl.BlockSpec((1, D), lambda e: (0, 0)),
        ],
        out_specs=pl.BlockSpec((BE, D), lambda e: (e, 0)),
        out_shape=jax.ShapeDtypeStruct((E, D), jnp.float32),
    )(edge_attr, w, b)


def _prep_body(x_ref, vn_ref, o_ref, o16_ref):
    h = x_ref[...] + vn_ref[0][None, :]
    o_ref[...] = h
    o16_ref[...] = h.astype(jnp.bfloat16)


def _prep(x, vn_emb):
    return pl.pallas_call(
        _prep_body,
        grid=(NGRID,),
        in_specs=[
            pl.BlockSpec((NB, D), lambda n: (n, 0)),
            pl.BlockSpec((L, D), lambda n: (0, 0)),
        ],
        out_specs=[pl.BlockSpec((NB, D), lambda n: (n, 0))] * 2,
        out_shape=[
            jax.ShapeDtypeStruct((N, D), jnp.float32),
            jax.ShapeDtypeStruct((N, D), jnp.bfloat16),
        ],
    )(x, vn_emb)


_SC_MESH = plsc.VectorSubcoreMesh(core_axis_name="c", subcore_axis_name="s")


@functools.partial(
    pl.kernel,
    out_type=jax.ShapeDtypeStruct((2, NPAD, D), jnp.float32),
    mesh=_SC_MESH,
    compiler_params=pltpu.CompilerParams(use_tc_tiling_on_sc=False,
                                         needs_layout_passes=False),
    scratch_types=[
        pltpu.VMEM((4, 2, 2, CK), jnp.int32),
        pltpu.VMEM((2, CK, D // 2), jnp.int32),
        pltpu.VMEM((2, CK, D), jnp.float32),
        pltpu.VMEM_SHARED((NPAD, D), jnp.float32),
        pltpu.SemaphoreType.DMA,
        pltpu.SemaphoreType.DMA,
        pltpu.SemaphoreType.DMA,
        pltpu.SemaphoreType.DMA,
        pltpu.SemaphoreType.DMA,
        pltpu.SemaphoreType.DMA,
    ],
)
def _sc_aggregate(h16_hbm, eemb_hbm, rec_hbm, zero_hbm, out_hbm,
                  rec_v, rows_v, emb_v, acc,
                  gs0, gs1, ss0, ss1, es0, es1):
    c = lax.axis_index("c")
    s = lax.axis_index("s")
    tile = c * 16 + s
    gsem = (gs0, gs1)
    ssem = (ss0, ss1)
    esem = (es0, es1)
    ebase = tile * EPT

    def rec_load(p, pb):
        pltpu.sync_copy(rec_hbm.at[tile, p], rec_v.at[pb])

    def gather_start(pb, w, rb):
        pltpu.async_copy(h16_hbm.at[rec_v.at[pb, w, 0]], rows_v.at[rb],
                         gsem[rb])

    def gather_wait(pb, w, rb):
        pltpu.make_async_copy(h16_hbm.at[rec_v.at[pb, w, 0]], rows_v.at[rb],
                              gsem[rb]).wait()

    def emb_start(j, eb):
        pltpu.async_copy(eemb_hbm.at[pl.ds(ebase + j * CK, CK)],
                         emb_v.at[eb], esem[eb])

    def emb_wait(j, eb):
        pltpu.make_async_copy(eemb_hbm.at[pl.ds(ebase + j * CK, CK)],
                              emb_v.at[eb], esem[eb]).wait()

    def scatter_start(pb, w, eb):
        pltpu.async_copy(emb_v.at[eb], acc.at[rec_v.at[pb, w, 1]],
                         ssem[eb], add=True)

    def scatter_drain(pb, w, eb):
        pltpu.make_async_copy(emb_v.at[eb], acc.at[rec_v.at[pb, w, 1]],
                              ssem[eb]).wait()

    rec_load(0, 0)
    rec_load(1, 1)
    gather_start(0, 0, 0)
    gather_start(0, 1, 1)
    emb_start(0, 0)
    pltpu.sync_copy(zero_hbm.at[pl.ds(s * RPS, RPS)],
                    acc.at[pl.ds(s * RPS, RPS)])
    plsc.subcore_barrier()

    def oct_body(jj, carry):
        for q in range(8):
            j = jj * 8 + q
            rb = q % 2
            nb = (q + 1) % 2
            pb_j = (q // 2) % 4
            w_j = q % 2
            if q % 2 == 0:
                pb_p = ((q + 6) // 2) % 4
                w_p = 1
            else:
                pb_p = ((q - 1) // 2) % 4
                w_p = 0
            if q > 0:
                scatter_drain(pb_p, w_p, nb)
            else:
                @pl.when(jj > 0)
                def _():
                    scatter_drain(pb_p, w_p, nb)
            if q == 7:
                @pl.when(jj < NOCT - 1)
                def _():
                    emb_start(j + 1, nb)
            else:
                emb_start(j + 1, nb)
            gather_wait(pb_j, w_j, rb)
            emb_wait(j, rb)

            def row_body(r, carry2):
                for g in range(D // 32):
                    hw = rows_v[rb, r, pl.ds(g * 16, 16)]
                    av = plsc.bitcast(hw << 16, jnp.float32)
                    bv = plsc.bitcast(hw & jnp.int32(-65536), jnp.float32)
                    sa = pl.ds(g * 32, 16)
                    sb = pl.ds(g * 32 + 16, 16)
                    emb_v[rb, r, sa] = jnp.maximum(av + emb_v[rb, r, sa], 0.0)
                    emb_v[rb, r, sb] = jnp.maximum(bv + emb_v[rb, r, sb], 0.0)
                return carry2

            lax.fori_loop(0, CK, row_body, 0, unroll=2)
            if q >= 6:
                @pl.when(jj < NOCT - 1)
                def _():
                    gather_start(((q + 2) // 2) % 4, w_j, rb)
            else:
                gather_start(((q + 2) // 2) % 4, w_j, rb)
            scatter_start(pb_j, w_j, rb)
            if q % 2 == 1:
                if q >= 5:
                    @pl.when(jj < NOCT - 1)
                    def _():
                        rec_load(4 * jj + (q + 3) // 2, ((q + 3) // 2) % 4)
                else:
                    rec_load(4 * jj + (q + 3) // 2, ((q + 3) // 2) % 4)
        return carry

    lax.fori_loop(0, NOCT, oct_body, 0, unroll=False)
    scatter_drain(3, 1, 1)
    plsc.subcore_barrier()
    pltpu.sync_copy(acc.at[pl.ds(s * RPS, RPS)],
                    out_hbm.at[c, pl.ds(s * RPS, RPS)])


def _layer_body(h_ref, agg_ref, w1_ref, w1p_ref, b1_ref, w2_ref, b2_ref,
                vn_ref, bt_ref, hout_ref, h16_ref, g_ref):
    aggp = agg_ref[0] + agg_ref[1]
    z = jnp.maximum(
        jnp.dot(h_ref[...], w1_ref[...], preferred_element_type=jnp.float32)
        + jnp.dot(aggp, w1p_ref[...], preferred_element_type=jnp.float32)
        + b1_ref[...], 0.0)
    h_new = (jnp.dot(z, w2_ref[...], preferred_element_type=jnp.float32)
             + b2_ref[...])
    h_cur = h_new + vn_ref[...]
    hout_ref[...] = h_cur
    h16_ref[...] = h_cur.astype(jnp.bfloat16)
    onehot = (bt_ref[0, 0, :][None, :]
              == lax.broadcasted_iota(jnp.int32, (G, NB), 0)
              ).astype(jnp.float32)

    @pl.when(pl.program_id(0) == 0)
    def _():
        g_ref[...] = jnp.zeros_like(g_ref)

    g_ref[...] += jnp.dot(onehot, h_new, preferred_element_type=jnp.float32)


def _layer(h_cur, agg2, w1, w1p, b1, w2, b2, vn_next, batch2):
    return pl.pallas_call(
        _layer_body,
        grid=(NGRID,),
        in_specs=[
            pl.BlockSpec((NB, D), lambda n: (n, 0)),
            pl.BlockSpec((2, NB, D), lambda n: (0, n, 0)),

            pl.BlockSpec((D, D), lambda n: (0, 0)),
            pl.BlockSpec((D, D), lambda n: (0, 0)),
            pl.BlockSpec((1, D), lambda n: (0, 0)),
            pl.BlockSpec((D, D), lambda n: (0, 0)),
            pl.BlockSpec((1, D), lambda n: (0, 0)),
            pl.BlockSpec((1, D), lambda n: (0, 0)),
            pl.BlockSpec((1, 1, NB), lambda n: (n, 0, 0)),
        ],
        out_specs=[
            pl.BlockSpec((NB, D), lambda n: (n, 0)),
            pl.BlockSpec((NB, D), lambda n: (n, 0)),
            pl.BlockSpec((G, D), lambda n: (0, 0)),
        ],
        out_shape=[
            jax.ShapeDtypeStruct((N, D), jnp.float32),
            jax.ShapeDtypeStruct((N, D), jnp.bfloat16),
            jax.ShapeDtypeStruct((G, D), jnp.float32),
        ],
    )(h_cur, agg2, w1, w1p, b1, w2, b2, vn_next, batch2)


def _layer_norm(h, g, b):
    mu = jnp.mean(h, axis=-1, keepdims=True)
    var = jnp.mean((h - mu) ** 2, axis=-1, keepdims=True)
    return (h - mu) * lax.rsqrt(var + 1e-5) * g + b


def _final_body(g0_ref, g1_ref, g2_ref, w0_ref, b0_ref, ln_g0_ref, ln_b0_ref,
                w1_ref, b1_ref, ln_g1_ref, ln_b1_ref, we_ref, be_ref,
                wo_ref, bo_ref, out_ref):
    q = (jnp.dot(g0_ref[...], w0_ref[pl.ds(0, D), :],
                 preferred_element_type=jnp.float32)
         + jnp.dot(g1_ref[...], w0_ref[pl.ds(D, D), :],
                   preferred_element_type=jnp.float32)
         + jnp.dot(g2_ref[...], w0_ref[pl.ds(2 * D, D), :],
                   preferred_element_type=jnp.float32)
         + b0_ref[...])
    q = jnp.maximum(_layer_norm(q, ln_g0_ref[...], ln_b0_ref[...]), 0.0)
    q = jnp.dot(q, w1_ref[...], preferred_element_type=jnp.float32) + b1_ref[...]
    q = jnp.maximum(_layer_norm(q, ln_g1_ref[...], ln_b1_ref[...]), 0.0)
    emb = jnp.dot(q, we_ref[...], preferred_element_type=jnp.float32) + be_ref[...]
    out_ref[...] = (jnp.dot(emb, wo_ref[...], preferred_element_type=jnp.float32)
                    + bo_ref[...])


def _final(g0, g1, g2, w0, b0, ln_g0, ln_b0, w1, b1, ln_g1, ln_b1,
           we, be, wo, bo):
    return pl.pallas_call(
        _final_body,
        out_shape=jax.ShapeDtypeStruct((G, 1), jnp.float32),
    )(g0, g1, g2, w0, b0, ln_g0, ln_b0, w1, b1, ln_g1, ln_b1, we, be, wo, bo)


def kernel(x, edge_index, edge_attr, batch, vn_emb, vn_W1, vn_b1, vn_g, vn_bt,
           vn_W2, vn_b2, agg_eW, agg_eb, agg_W1, agg_b1, agg_W2, agg_b2,
           lin_W0, lin_b0, lin_g0, lin_bt0, lin_W1, lin_b1, lin_g1, lin_bt1,
           emb_W, emb_b, out_W, out_b):
    src_r = edge_index[0].reshape(NT, CH // 2, 2, 1, CK).astype(jnp.int32)
    dst_r = edge_index[1].reshape(NT, CH // 2, 2, 1, CK).astype(jnp.int32)
    rec = jnp.concatenate([src_r, dst_r], axis=3)
    batch2 = batch.reshape(NGRID, 1, NB).astype(jnp.int32)
    zero_nd = jnp.zeros((NPAD, D), jnp.float32)
    perm = jnp.array(PERM, jnp.int32)

    def pack_rows(h16):
        return lax.bitcast_convert_type(h16.reshape(N, D // 2, 2), jnp.int32)

    eW_p = agg_eW[:, :, perm]
    eb_p = agg_eb[:, perm]
    eemb_i = _edge_emb(edge_attr, eW_p[0:1], eb_p[0:1])
    h_cur, h16 = _prep(x, vn_emb)

    gs = []
    for i in range(L):
        agg2 = _sc_aggregate(pack_rows(h16), eemb_i, rec, zero_nd)
        if i + 1 < L:
            eemb_i = _edge_emb(edge_attr, eW_p[i + 1:i + 2],
                               eb_p[i + 1:i + 2])
        if i + 1 < L:
            vn_next = vn_emb[i + 1].reshape(1, D)
        else:
            vn_next = jnp.zeros((1, D), jnp.float32)
        h_cur, h16, g_i = _layer(h_cur, agg2, agg_W1[i], agg_W1[i][perm, :],
                                 agg_b1[i].reshape(1, D), agg_W2[i],
                                 agg_b2[i].reshape(1, D), vn_next, batch2)
        gs.append(g_i)

    return _final(gs[0], gs[1], gs[2], lin_W0, lin_b0.reshape(1, D),
                  lin_g0.reshape(1, D), lin_bt0.reshape(1, D), lin_W1,
                  lin_b1.reshape(1, D), lin_g1.reshape(1, D),
                  lin_bt1.reshape(1, D), emb_W, emb_b.reshape(1, D),
                  out_W, out_b.reshape(1, 1))
